# baseline (speedup 1.0000x reference)
import functools

import jax
import jax.numpy as jnp
from jax import lax
from jax.experimental import pallas as pl
from jax.experimental.pallas import tpu as pltpu
from jax.experimental.pallas import tpu_sc as plsc

EMBED_DIM = 64
BATCH = 4096
SEQ = 200
NC = 2
NS = 16
NW = NC * NS
NCHUNK = 4
B_CHUNK = BATCH // NCHUNK
ROWS_PER_W = B_CHUNK // NW
NSLOT = 4
NGROUPS = ROWS_PER_W // NSLOT

_mesh = plsc.VectorSubcoreMesh(core_axis_name="c", subcore_axis_name="s")


@functools.partial(
    pl.kernel,
    out_type=jax.ShapeDtypeStruct((B_CHUNK, SEQ, EMBED_DIM), jnp.float32),
    mesh=_mesh,
    scratch_types=[
        pltpu.VMEM((ROWS_PER_W, SEQ), jnp.int32),
        pltpu.VMEM((NSLOT, SEQ, EMBED_DIM), jnp.float32),
    ]
    + [pltpu.SemaphoreType.DMA] * (2 * NSLOT),
    compiler_params=pltpu.CompilerParams(use_tc_tiling_on_sc=False),
)
def _embed_gather(table_hbm, x_hbm, out_hbm, idx_v, rows_v, *sems):
    gsem = sems[:NSLOT]
    wsem = sems[NSLOT:]
    wid = lax.axis_index("s") * NC + lax.axis_index("c")
    b0 = wid * ROWS_PER_W
    pltpu.sync_copy(x_hbm.at[pl.ds(b0, ROWS_PER_W)], idx_v)

    for s in range(NSLOT):
        pltpu.async_copy(table_hbm.at[idx_v.at[s]], rows_v.at[s], gsem[s])

    @pl.loop(0, NGROUPS)
    def _ring(grp):
        i0 = grp * NSLOT
        for s in range(NSLOT):
            pltpu.make_async_copy(
                table_hbm.at[idx_v.at[i0 + s]], rows_v.at[s], gsem[s]
            ).wait()
            pltpu.async_copy(rows_v.at[s], out_hbm.at[b0 + i0 + s], wsem[s])
        for s in range(NSLOT):
            pltpu.make_async_copy(
                rows_v.at[s], out_hbm.at[b0 + i0 + s], wsem[s]
            ).wait()

            @pl.when(grp < NGROUPS - 1)
            def _():
                pltpu.async_copy(
                    table_hbm.at[idx_v.at[i0 + NSLOT + s]], rows_v.at[s], gsem[s]
                )


def kernel(x, table):
    xi = x.astype(jnp.int32)
    chunks = [
        _embed_gather(table, xi[c * B_CHUNK : (c + 1) * B_CHUNK])
        for c in range(NCHUNK)
    ]
    return jnp.concatenate(chunks, axis=0)


# speedup vs baseline: 1.7486x; 1.7486x over previous
import functools

import jax
import jax.numpy as jnp
from jax import lax
from jax.experimental import pallas as pl
from jax.experimental.pallas import tpu as pltpu
from jax.experimental.pallas import tpu_sc as plsc

EMBED_DIM = 64
N_VOCAB = 100000
BATCH = 4096
SEQ = 200
NC = 2
NS = 16
NW = NC * NS
ROWS_PER_W = BATCH // NW
NSLOT = 3
NGROUPS = ROWS_PER_W // NSLOT
IDX_PER_W = ROWS_PER_W * SEQ

_mesh = plsc.VectorSubcoreMesh(core_axis_name="c", subcore_axis_name="s")


@functools.partial(
    pl.kernel,
    out_type=jax.ShapeDtypeStruct((BATCH, SEQ, 2 * EMBED_DIM), jnp.float32),
    mesh=_mesh,
    scratch_types=[
        pltpu.VMEM((IDX_PER_W,), jnp.int32),
        pltpu.VMEM((NSLOT, SEQ, 2 * EMBED_DIM), jnp.float32),
    ]
    + [pltpu.SemaphoreType.DMA] * (2 * NSLOT),
)
def _embed_gather(table_hbm, x_hbm, out_hbm, idx_v, rows_v, *sems):
    gsem = sems[:NSLOT]
    wsem = sems[NSLOT:]
    wid = lax.axis_index("s") * NC + lax.axis_index("c")
    b0 = wid * ROWS_PER_W

    pltpu.sync_copy(x_hbm.at[pl.ds(b0 * SEQ, IDX_PER_W)], idx_v)

    def gather(i, s):
        pltpu.async_copy(
            table_hbm.at[idx_v.at[pl.ds(i * SEQ, SEQ)]], rows_v.at[s], gsem[s]
        )

    def gather_wait(i, s):
        pltpu.make_async_copy(
            table_hbm.at[idx_v.at[pl.ds(i * SEQ, SEQ)]], rows_v.at[s], gsem[s]
        ).wait()

    def write(i, s):
        pltpu.async_copy(rows_v.at[s], out_hbm.at[b0 + i], wsem[s])

    def write_wait(i, s):
        pltpu.make_async_copy(rows_v.at[s], out_hbm.at[b0 + i], wsem[s]).wait()

    for s in range(NSLOT):
        gather(s, s)

    @pl.loop(0, NGROUPS)
    def _ring(grp):
        i0 = grp * NSLOT
        for s in range(NSLOT):
            gather_wait(i0 + s, s)
            write(i0 + s, s)
        for s in range(NSLOT):
            write_wait(i0 + s, s)

            @pl.when(grp < NGROUPS - 1)
            def _():
                gather(i0 + NSLOT + s, s)

    for i in range(NGROUPS * NSLOT, ROWS_PER_W):
        gather(i, i % NSLOT)
    for i in range(NGROUPS * NSLOT, ROWS_PER_W):
        gather_wait(i, i % NSLOT)
        write(i, i % NSLOT)
    for i in range(NGROUPS * NSLOT, ROWS_PER_W):
        write_wait(i, i % NSLOT)


TBLK = 1024


def _widen_body(t_ref, o_ref):
    tt = t_ref[...].T
    o_ref[...] = jnp.concatenate([tt, tt], axis=1)


def _table_widen(table_t):
    grid = pl.cdiv(N_VOCAB, TBLK)
    return pl.pallas_call(
        _widen_body,
        out_shape=jax.ShapeDtypeStruct((N_VOCAB, 2 * EMBED_DIM), jnp.float32),
        grid=(grid,),
        in_specs=[pl.BlockSpec((EMBED_DIM, TBLK), lambda i: (0, i))],
        out_specs=pl.BlockSpec((TBLK, 2 * EMBED_DIM), lambda i: (i, 0)),
    )(table_t)


def kernel(x, table):
    table3 = _table_widen(table.T)
    wide = _embed_gather(table3, x.astype(jnp.int32).reshape(-1))
    return wide[:, :, :EMBED_DIM]


# speedup vs baseline: 1.7490x; 1.0002x over previous
import functools

import jax
import jax.numpy as jnp
from jax import lax
from jax.experimental import pallas as pl
from jax.experimental.pallas import tpu as pltpu
from jax.experimental.pallas import tpu_sc as plsc

EMBED_DIM = 64
N_VOCAB = 100000
BATCH = 4096
SEQ = 200
NC = 2
NS = 16
NW = NC * NS
ROWS_PER_W = BATCH // NW
NSLOT = 4
NGROUPS = ROWS_PER_W // NSLOT
IDX_PER_W = ROWS_PER_W * SEQ

_mesh = plsc.VectorSubcoreMesh(core_axis_name="c", subcore_axis_name="s")


@functools.partial(
    pl.kernel,
    out_type=jax.ShapeDtypeStruct((BATCH, SEQ, 2 * EMBED_DIM), jnp.float32),
    mesh=_mesh,
    scratch_types=[
        pltpu.VMEM((IDX_PER_W,), jnp.int32),
        pltpu.VMEM((NSLOT, SEQ, 2 * EMBED_DIM), jnp.float32),
    ]
    + [pltpu.SemaphoreType.DMA] * (2 * NSLOT),
)
def _embed_gather(table_hbm, x_hbm, out_hbm, idx_v, rows_v, *sems):
    gsem = sems[:NSLOT]
    wsem = sems[NSLOT:]
    wid = lax.axis_index("s") * NC + lax.axis_index("c")
    b0 = wid * ROWS_PER_W

    pltpu.sync_copy(x_hbm.at[pl.ds(b0 * SEQ, IDX_PER_W)], idx_v)

    def gather(i, s):
        pltpu.async_copy(
            table_hbm.at[idx_v.at[pl.ds(i * SEQ, SEQ)]], rows_v.at[s], gsem[s]
        )

    def gather_wait(i, s):
        pltpu.make_async_copy(
            table_hbm.at[idx_v.at[pl.ds(i * SEQ, SEQ)]], rows_v.at[s], gsem[s]
        ).wait()

    def write(i, s):
        pltpu.async_copy(rows_v.at[s], out_hbm.at[b0 + i], wsem[s])

    def write_wait(i, s):
        pltpu.make_async_copy(rows_v.at[s], out_hbm.at[b0 + i], wsem[s]).wait()

    for s in range(NSLOT):
        gather(s, s)

    @pl.loop(0, NGROUPS)
    def _ring(grp):
        i0 = grp * NSLOT
        for s in range(NSLOT):
            gather_wait(i0 + s, s)
            write(i0 + s, s)
        for s in range(NSLOT):
            write_wait(i0 + s, s)

            @pl.when(grp < NGROUPS - 1)
            def _():
                gather(i0 + NSLOT + s, s)

    for i in range(NGROUPS * NSLOT, ROWS_PER_W):
        gather(i, i % NSLOT)
    for i in range(NGROUPS * NSLOT, ROWS_PER_W):
        gather_wait(i, i % NSLOT)
        write(i, i % NSLOT)
    for i in range(NGROUPS * NSLOT, ROWS_PER_W):
        write_wait(i, i % NSLOT)


TBLK = 1024


def _widen_body(t_ref, o_ref):
    tt = t_ref[...].T
    o_ref[...] = jnp.concatenate([tt, tt], axis=1)


def _table_widen(table_t):
    grid = pl.cdiv(N_VOCAB, TBLK)
    return pl.pallas_call(
        _widen_body,
        out_shape=jax.ShapeDtypeStruct((N_VOCAB, 2 * EMBED_DIM), jnp.float32),
        grid=(grid,),
        in_specs=[pl.BlockSpec((EMBED_DIM, TBLK), lambda i: (0, i))],
        out_specs=pl.BlockSpec((TBLK, 2 * EMBED_DIM), lambda i: (i, 0)),
    )(table_t)


def kernel(x, table):
    table3 = _table_widen(table.T)
    wide = _embed_gather(table3, x.astype(jnp.int32).reshape(-1))
    return wide[:, :, :EMBED_DIM]


# speedup vs baseline: 2.5385x; 1.4514x over previous
import functools

import jax
import jax.numpy as jnp
from jax import lax
from jax.experimental import pallas as pl
from jax.experimental.pallas import tpu as pltpu
from jax.experimental.pallas import tpu_sc as plsc

EMBED_DIM = 64
N_VOCAB = 100000
BATCH = 4096
SEQ = 200
NC = 2
NS = 16
NW = NC * NS
ROWS_PER_W = BATCH // NW
NSLOT = 4
NGROUPS = ROWS_PER_W // NSLOT
IDX_PER_W = ROWS_PER_W * SEQ

_mesh = plsc.VectorSubcoreMesh(core_axis_name="c", subcore_axis_name="s")


@functools.partial(
    pl.kernel,
    out_type=jax.ShapeDtypeStruct((BATCH, SEQ, 2 * EMBED_DIM), jnp.float32),
    mesh=_mesh,
    scratch_types=[
        pltpu.VMEM((IDX_PER_W,), jnp.int32),
        pltpu.VMEM((NSLOT, SEQ, EMBED_DIM), jnp.float32),
    ]
    + [pltpu.SemaphoreType.DMA] * (2 * NSLOT),
    compiler_params=pltpu.CompilerParams(use_tc_tiling_on_sc=False),
)
def _embed_gather(table_hbm, x_hbm, out_hbm, idx_v, rows_v, *sems):
    gsem = sems[:NSLOT]
    wsem = sems[NSLOT:]
    tbl = table_hbm
    wid = lax.axis_index("s") * NC + lax.axis_index("c")
    b0 = wid * ROWS_PER_W

    pltpu.sync_copy(x_hbm.at[pl.ds(b0 * SEQ, IDX_PER_W)], idx_v)

    def gather(i, s):
        pltpu.async_copy(
            tbl.at[idx_v.at[pl.ds(i * SEQ, SEQ)]], rows_v.at[s], gsem[s]
        )

    def gather_wait(i, s):
        pltpu.make_async_copy(
            tbl.at[idx_v.at[pl.ds(i * SEQ, SEQ)]], rows_v.at[s], gsem[s]
        ).wait()

    def write(i, s):
        pltpu.async_copy(
            rows_v.at[s], out_hbm.at[b0 + i, :, pl.ds(0, EMBED_DIM)], wsem[s]
        )

    def write_wait(i, s):
        pltpu.make_async_copy(
            rows_v.at[s], out_hbm.at[b0 + i, :, pl.ds(0, EMBED_DIM)], wsem[s]
        ).wait()

    for s in range(NSLOT):
        gather(s, s)

    @pl.loop(0, NGROUPS)
    def _ring(grp):
        i0 = grp * NSLOT
        for s in range(NSLOT):
            gather_wait(i0 + s, s)
            write(i0 + s, s)
        for s in range(NSLOT):
            write_wait(i0 + s, s)

            @pl.when(grp < NGROUPS - 1)
            def _():
                gather(i0 + NSLOT + s, s)


TBLK = 1024
NFOLD = 49
VPAD = NFOLD * TBLK


def _fold_body(lo_ref, hi_ref, o_ref):
    o_ref[:, :EMBED_DIM] = lo_ref[...].T
    o_ref[:, EMBED_DIM:] = hi_ref[...].T


def _table_fold(table_t):
    return pl.pallas_call(
        _fold_body,
        out_shape=jax.ShapeDtypeStruct((VPAD, 2 * EMBED_DIM), jnp.float32),
        grid=(NFOLD,),
        in_specs=[
            pl.BlockSpec((EMBED_DIM, TBLK), lambda i: (0, i)),
            pl.BlockSpec((EMBED_DIM, TBLK), lambda i: (0, i + NFOLD)),
        ],
        out_specs=pl.BlockSpec((TBLK, 2 * EMBED_DIM), lambda i: (i, 0)),
    )(table_t, table_t)


def kernel(x, table):
    table2 = _table_fold(table.T).reshape(2 * VPAD, EMBED_DIM)
    xi = x.astype(jnp.int32)
    xr = jnp.where(xi < VPAD, 2 * xi, 2 * xi - (2 * VPAD - 1))
    wide = _embed_gather(table2, xr.reshape(-1))
    return wide[:, :, :EMBED_DIM]
